# Initial kernel scaffold; baseline (speedup 1.0000x reference)
import jax
import jax.numpy as jnp
from jax.experimental import pallas as pl


def kernel(x, edge_index, W1, b1, W2, b2):
    raise NotImplementedError("write your pallas kernel here")


# speedup vs baseline: 9.1168x; 9.1168x over previous
import functools

import jax
import jax.numpy as jnp
from jax import lax
from jax.experimental import pallas as pl
from jax.experimental.pallas import tpu as pltpu
from jax.experimental.pallas import tpu_sc as plsc

N = 10000
D_IN = 128
D_MID = 256
D_FIN = 128
E = 320000

LANES = 16
NC = 2
NS = 16

EPR = 128
ROWS_PAD = 2560
E_PAD = ROWS_PAD * EPR
RPT = ROWS_PAD // NS
IDXC = 32
ACC_ROWS = 10008
NPT = N // NS
DEG_EPT = E_PAD // (NC * NS)
CNT_ROWS = 10016

BN = 1000
GRID = N // BN

_MESH = plsc.VectorSubcoreMesh(
    core_axis_name="c", subcore_axis_name="s", num_cores=NC, num_subcores=NS)


def _deg_body(dst_hbm, out_hbm, dst_v, cnt_v):
    c = lax.axis_index("c")
    s = lax.axis_index("s")
    wid = s * NC + c
    pltpu.sync_copy(dst_hbm.at[pl.ds(wid * DEG_EPT, DEG_EPT)], dst_v)
    zeros16 = jnp.zeros((LANES,), jnp.float32)
    ones16 = jnp.ones((LANES,), jnp.float32)

    def zloop(i, carry):
        cnt_v[pl.ds(i * LANES, LANES)] = zeros16
        return carry

    lax.fori_loop(0, CNT_ROWS // LANES, zloop, 0)

    def aloop(j, carry):
        idx = dst_v[pl.ds(j * LANES, LANES)]
        plsc.addupdate_scatter(cnt_v, [idx], ones16)
        return carry

    lax.fori_loop(0, DEG_EPT // LANES, aloop, 0)
    pltpu.sync_copy(cnt_v.at[pl.ds(0, N)], out_hbm.at[wid])


_deg_call = pl.kernel(
    _deg_body,
    out_type=jax.ShapeDtypeStruct((NC * NS, N), jnp.float32),
    mesh=_MESH,
    compiler_params=pltpu.CompilerParams(
        needs_layout_passes=False, use_tc_tiling_on_sc=False),
    scratch_types=[
        pltpu.VMEM((DEG_EPT,), jnp.int32),
        pltpu.VMEM((CNT_ROWS,), jnp.float32),
    ],
)


def _make_agg(d_half):
    def body(glo, ghi, src_hbm, dst_hbm, zeros_hbm, out_hbm,
             idx_s, idx_d, buf, acc_sh, sem):
        c = lax.axis_index("c")
        s = lax.axis_index("s")
        pltpu.sync_copy(zeros_hbm, acc_sh.at[pl.ds(s * NPT, NPT)])
        plsc.subcore_barrier()

        def chunk_loop(ci, carry):
            row0 = s * RPT + ci * IDXC
            pltpu.sync_copy(src_hbm.at[pl.ds(row0, IDXC)], idx_s)
            pltpu.sync_copy(dst_hbm.at[pl.ds(row0, IDXC)], idx_d)

            def loop(j, c2):
                @pl.when(c == 0)
                def _():
                    pltpu.async_copy(glo.at[idx_s.at[j]], buf, sem).wait()

                @pl.when(c == 1)
                def _():
                    pltpu.async_copy(ghi.at[idx_s.at[j]], buf, sem).wait()

                pltpu.sync_copy(buf, acc_sh.at[idx_d.at[j]], add=True)
                return c2

            lax.fori_loop(0, IDXC, loop, carry)
            return carry

        lax.fori_loop(0, RPT // IDXC, chunk_loop, 0)
        plsc.subcore_barrier()
        pltpu.sync_copy(
            acc_sh.at[pl.ds(s * NPT, NPT)],
            out_hbm.at[pl.ds(s * NPT, NPT), pl.ds(c * d_half, d_half)])

    return pl.kernel(
        body,
        out_type=jax.ShapeDtypeStruct((N, 2 * d_half), jnp.float32),
        mesh=_MESH,
        compiler_params=pltpu.CompilerParams(use_tc_tiling_on_sc=False),
        scratch_types=[
            pltpu.VMEM((IDXC, EPR), jnp.int32),
            pltpu.VMEM((IDXC, EPR), jnp.int32),
            pltpu.VMEM((EPR, d_half), jnp.float32),
            pltpu.VMEM_SHARED((ACC_ROWS, d_half), jnp.float32),
            pltpu.SemaphoreType.DMA,
        ],
    )


_agg_l1 = _make_agg(128)
_agg_l2 = _make_agg(64)


def _tc_a_body(x_ref, w_ref, degp_ref, glo_ref, ghi_ref, dinv_ref):
    deg = jnp.sum(degp_ref[...], axis=1, keepdims=True) + 1.0
    dinv = lax.rsqrt(deg)
    h = jnp.dot(x_ref[...], w_ref[...], preferred_element_type=jnp.float32)
    g = h * dinv
    glo_ref[...] = g[:, :D_MID // 2]
    ghi_ref[...] = g[:, D_MID // 2:]
    dinv_ref[...] = dinv


_tc_a = pl.pallas_call(
    _tc_a_body,
    grid=(GRID,),
    in_specs=[
        pl.BlockSpec((BN, D_IN), lambda i: (i, 0)),
        pl.BlockSpec((D_IN, D_MID), lambda i: (0, 0)),
        pl.BlockSpec((BN, NC * NS), lambda i: (i, 0)),
    ],
    out_specs=[
        pl.BlockSpec((BN, D_MID // 2), lambda i: (i, 0)),
        pl.BlockSpec((BN, D_MID // 2), lambda i: (i, 0)),
        pl.BlockSpec((BN, 1), lambda i: (i, 0)),
    ],
    out_shape=[
        jax.ShapeDtypeStruct((N, D_MID // 2), jnp.float32),
        jax.ShapeDtypeStruct((N, D_MID // 2), jnp.float32),
        jax.ShapeDtypeStruct((N, 1), jnp.float32),
    ],
)


def _tc_b_body(acc_ref, glo_ref, ghi_ref, dinv_ref, b1_ref, w2_ref,
               g2lo_ref, g2hi_ref):
    g = jnp.concatenate([glo_ref[...], ghi_ref[...]], axis=1)
    dinv = dinv_ref[...]
    h1 = jnp.maximum((acc_ref[...] + g) * dinv + b1_ref[...][None, :], 0.0)
    h2 = jnp.dot(h1, w2_ref[...], preferred_element_type=jnp.float32)
    g2 = h2 * dinv
    g2lo_ref[...] = g2[:, :D_FIN // 2]
    g2hi_ref[...] = g2[:, D_FIN // 2:]


_tc_b = pl.pallas_call(
    _tc_b_body,
    grid=(GRID,),
    in_specs=[
        pl.BlockSpec((BN, D_MID), lambda i: (i, 0)),
        pl.BlockSpec((BN, D_MID // 2), lambda i: (i, 0)),
        pl.BlockSpec((BN, D_MID // 2), lambda i: (i, 0)),
        pl.BlockSpec((BN, 1), lambda i: (i, 0)),
        pl.BlockSpec((D_MID,), lambda i: (0,)),
        pl.BlockSpec((D_MID, D_FIN), lambda i: (0, 0)),
    ],
    out_specs=[
        pl.BlockSpec((BN, D_FIN // 2), lambda i: (i, 0)),
        pl.BlockSpec((BN, D_FIN // 2), lambda i: (i, 0)),
    ],
    out_shape=[
        jax.ShapeDtypeStruct((N, D_FIN // 2), jnp.float32),
        jax.ShapeDtypeStruct((N, D_FIN // 2), jnp.float32),
    ],
)


def _tc_c_body(acc_ref, glo_ref, ghi_ref, dinv_ref, b2_ref, out_ref):
    g = jnp.concatenate([glo_ref[...], ghi_ref[...]], axis=1)
    out = (acc_ref[...] + g) * dinv_ref[...] + b2_ref[...][None, :]
    out_ref[...] = jnp.maximum(out, 0.0)


_tc_c = pl.pallas_call(
    _tc_c_body,
    grid=(GRID,),
    in_specs=[
        pl.BlockSpec((BN, D_FIN), lambda i: (i, 0)),
        pl.BlockSpec((BN, D_FIN // 2), lambda i: (i, 0)),
        pl.BlockSpec((BN, D_FIN // 2), lambda i: (i, 0)),
        pl.BlockSpec((BN, 1), lambda i: (i, 0)),
        pl.BlockSpec((D_FIN,), lambda i: (0,)),
    ],
    out_specs=pl.BlockSpec((BN, D_FIN), lambda i: (i, 0)),
    out_shape=jax.ShapeDtypeStruct((N, D_FIN), jnp.float32),
)


@jax.jit
def kernel(x, edge_index, W1, b1, W2, b2):
    src = edge_index[0]
    dst = edge_index[1]
    npad = E_PAD - E
    src_pad = jnp.concatenate([src, jnp.zeros((npad,), jnp.int32)])
    dst_pad = jnp.concatenate([dst, jnp.full((npad,), N, jnp.int32)])
    src2d = src_pad.reshape(ROWS_PAD, EPR)
    dst2d = dst_pad.reshape(ROWS_PAD, EPR)
    z1 = jnp.zeros((NPT, D_MID // 2), jnp.float32)
    z2 = jnp.zeros((NPT, D_FIN // 2), jnp.float32)

    degp = _deg_call(dst_pad)
    g1lo, g1hi, dinv = _tc_a(x, W1, degp.T)
    acc1 = _agg_l1(g1lo, g1hi, src2d, dst2d, z1)
    g2lo, g2hi = _tc_b(acc1, g1lo, g1hi, dinv, b1, W2)
    acc2 = _agg_l2(g2lo, g2hi, src2d, dst2d, z2)
    return _tc_c(acc2, g2lo, g2hi, dinv, b2)


# speedup vs baseline: 11.1119x; 1.2188x over previous
import functools

import jax
import jax.numpy as jnp
from jax import lax
from jax.experimental import pallas as pl
from jax.experimental.pallas import tpu as pltpu
from jax.experimental.pallas import tpu_sc as plsc

N = 10000
D_IN = 128
D_MID = 256
D_FIN = 128
E = 320000

LANES = 16
NC = 2
NS = 16

EPR = 128
ROWS_PAD = 2560
E_PAD = ROWS_PAD * EPR
RPT = ROWS_PAD // NS
IDXC = 32
ACC_ROWS = 10008
NPT = N // NS
DEG_EPT = E_PAD // (NC * NS)
CNT_ROWS = 10016

BN = 1000
GRID = N // BN

_MESH = plsc.VectorSubcoreMesh(
    core_axis_name="c", subcore_axis_name="s", num_cores=NC, num_subcores=NS)


def _deg_body(dst_hbm, out_hbm, dst_v, cnt_v):
    c = lax.axis_index("c")
    s = lax.axis_index("s")
    wid = s * NC + c
    pltpu.sync_copy(dst_hbm.at[pl.ds(wid * DEG_EPT, DEG_EPT)], dst_v)
    zeros16 = jnp.zeros((LANES,), jnp.float32)
    ones16 = jnp.ones((LANES,), jnp.float32)

    def zloop(i, carry):
        cnt_v[pl.ds(i * LANES, LANES)] = zeros16
        return carry

    lax.fori_loop(0, CNT_ROWS // LANES, zloop, 0)

    def aloop(j, carry):
        idx = dst_v[pl.ds(j * LANES, LANES)]
        plsc.addupdate_scatter(cnt_v, [idx], ones16)
        return carry

    lax.fori_loop(0, DEG_EPT // LANES, aloop, 0)
    pltpu.sync_copy(cnt_v.at[pl.ds(0, N)], out_hbm.at[wid])


_deg_call = pl.kernel(
    _deg_body,
    out_type=jax.ShapeDtypeStruct((NC * NS, N), jnp.float32),
    mesh=_MESH,
    compiler_params=pltpu.CompilerParams(
        needs_layout_passes=False, use_tc_tiling_on_sc=False),
    scratch_types=[
        pltpu.VMEM((DEG_EPT,), jnp.int32),
        pltpu.VMEM((CNT_ROWS,), jnp.float32),
    ],
)


def _make_agg(d_half):
    def body(glo, ghi, src_hbm, dst_hbm, zeros_hbm, out_hbm,
             idx_s, idx_d, buf0, buf1, acc_sh, sem0, sem1):
        c = lax.axis_index("c")
        s = lax.axis_index("s")
        pltpu.sync_copy(zeros_hbm, acc_sh.at[pl.ds(s * NPT, NPT)])
        plsc.subcore_barrier()

        def gstart(j, buf, sem):
            @pl.when(c == 0)
            def _():
                pltpu.async_copy(glo.at[idx_s.at[j]], buf, sem)

            @pl.when(c == 1)
            def _():
                pltpu.async_copy(ghi.at[idx_s.at[j]], buf, sem)

        def gwait(buf, sem):
            pltpu.make_async_copy(glo.at[idx_s.at[0]], buf, sem).wait()

        def scat(j, buf):
            pltpu.sync_copy(buf, acc_sh.at[idx_d.at[j]], add=True)

        def chunk_loop(ci, carry):
            row0 = s * RPT + ci * IDXC
            pltpu.sync_copy(src_hbm.at[pl.ds(row0, IDXC)], idx_s)
            pltpu.sync_copy(dst_hbm.at[pl.ds(row0, IDXC)], idx_d)
            gstart(0, buf0, sem0)

            def loop(j, c2):
                r = 2 * j
                gstart(r + 1, buf1, sem1)
                gwait(buf0, sem0)
                scat(r, buf0)

                @pl.when(j < IDXC // 2 - 1)
                def _():
                    gstart(r + 2, buf0, sem0)

                gwait(buf1, sem1)
                scat(r + 1, buf1)
                return c2

            lax.fori_loop(0, IDXC // 2, loop, carry)
            return carry

        lax.fori_loop(0, RPT // IDXC, chunk_loop, 0)
        plsc.subcore_barrier()
        pltpu.sync_copy(
            acc_sh.at[pl.ds(s * NPT, NPT)],
            out_hbm.at[pl.ds(s * NPT, NPT), pl.ds(c * d_half, d_half)])

    return pl.kernel(
        body,
        out_type=jax.ShapeDtypeStruct((N, 2 * d_half), jnp.float32),
        mesh=_MESH,
        compiler_params=pltpu.CompilerParams(use_tc_tiling_on_sc=False),
        scratch_types=[
            pltpu.VMEM((IDXC, EPR), jnp.int32),
            pltpu.VMEM((IDXC, EPR), jnp.int32),
            pltpu.VMEM((EPR, d_half), jnp.float32),
            pltpu.VMEM((EPR, d_half), jnp.float32),
            pltpu.VMEM_SHARED((ACC_ROWS, d_half), jnp.float32),
            pltpu.SemaphoreType.DMA,
            pltpu.SemaphoreType.DMA,
        ],
    )


_agg_l1 = _make_agg(128)
_agg_l2 = _make_agg(64)


def _tc_a_body(x_ref, w_ref, degp_ref, glo_ref, ghi_ref, dinv_ref):
    deg = jnp.sum(degp_ref[...], axis=1, keepdims=True) + 1.0
    dinv = lax.rsqrt(deg)
    h = jnp.dot(x_ref[...], w_ref[...], preferred_element_type=jnp.float32)
    g = h * dinv
    glo_ref[...] = g[:, :D_MID // 2]
    ghi_ref[...] = g[:, D_MID // 2:]
    dinv_ref[...] = dinv


_tc_a = pl.pallas_call(
    _tc_a_body,
    grid=(GRID,),
    in_specs=[
        pl.BlockSpec((BN, D_IN), lambda i: (i, 0)),
        pl.BlockSpec((D_IN, D_MID), lambda i: (0, 0)),
        pl.BlockSpec((BN, NC * NS), lambda i: (i, 0)),
    ],
    out_specs=[
        pl.BlockSpec((BN, D_MID // 2), lambda i: (i, 0)),
        pl.BlockSpec((BN, D_MID // 2), lambda i: (i, 0)),
        pl.BlockSpec((BN, 1), lambda i: (i, 0)),
    ],
    out_shape=[
        jax.ShapeDtypeStruct((N, D_MID // 2), jnp.float32),
        jax.ShapeDtypeStruct((N, D_MID // 2), jnp.float32),
        jax.ShapeDtypeStruct((N, 1), jnp.float32),
    ],
)


def _tc_b_body(acc_ref, glo_ref, ghi_ref, dinv_ref, b1_ref, w2_ref,
               g2lo_ref, g2hi_ref):
    g = jnp.concatenate([glo_ref[...], ghi_ref[...]], axis=1)
    dinv = dinv_ref[...]
    h1 = jnp.maximum((acc_ref[...] + g) * dinv + b1_ref[...][None, :], 0.0)
    h2 = jnp.dot(h1, w2_ref[...], preferred_element_type=jnp.float32)
    g2 = h2 * dinv
    g2lo_ref[...] = g2[:, :D_FIN // 2]
    g2hi_ref[...] = g2[:, D_FIN // 2:]


_tc_b = pl.pallas_call(
    _tc_b_body,
    grid=(GRID,),
    in_specs=[
        pl.BlockSpec((BN, D_MID), lambda i: (i, 0)),
        pl.BlockSpec((BN, D_MID // 2), lambda i: (i, 0)),
        pl.BlockSpec((BN, D_MID // 2), lambda i: (i, 0)),
        pl.BlockSpec((BN, 1), lambda i: (i, 0)),
        pl.BlockSpec((D_MID,), lambda i: (0,)),
        pl.BlockSpec((D_MID, D_FIN), lambda i: (0, 0)),
    ],
    out_specs=[
        pl.BlockSpec((BN, D_FIN // 2), lambda i: (i, 0)),
        pl.BlockSpec((BN, D_FIN // 2), lambda i: (i, 0)),
    ],
    out_shape=[
        jax.ShapeDtypeStruct((N, D_FIN // 2), jnp.float32),
        jax.ShapeDtypeStruct((N, D_FIN // 2), jnp.float32),
    ],
)


def _tc_c_body(acc_ref, glo_ref, ghi_ref, dinv_ref, b2_ref, out_ref):
    g = jnp.concatenate([glo_ref[...], ghi_ref[...]], axis=1)
    out = (acc_ref[...] + g) * dinv_ref[...] + b2_ref[...][None, :]
    out_ref[...] = jnp.maximum(out, 0.0)


_tc_c = pl.pallas_call(
    _tc_c_body,
    grid=(GRID,),
    in_specs=[
        pl.BlockSpec((BN, D_FIN), lambda i: (i, 0)),
        pl.BlockSpec((BN, D_FIN // 2), lambda i: (i, 0)),
        pl.BlockSpec((BN, D_FIN // 2), lambda i: (i, 0)),
        pl.BlockSpec((BN, 1), lambda i: (i, 0)),
        pl.BlockSpec((D_FIN,), lambda i: (0,)),
    ],
    out_specs=pl.BlockSpec((BN, D_FIN), lambda i: (i, 0)),
    out_shape=jax.ShapeDtypeStruct((N, D_FIN), jnp.float32),
)


@jax.jit
def kernel(x, edge_index, W1, b1, W2, b2):
    src = edge_index[0]
    dst = edge_index[1]
    npad = E_PAD - E
    src_pad = jnp.concatenate([src, jnp.zeros((npad,), jnp.int32)])
    dst_pad = jnp.concatenate([dst, jnp.full((npad,), N, jnp.int32)])
    src2d = src_pad.reshape(ROWS_PAD, EPR)
    dst2d = dst_pad.reshape(ROWS_PAD, EPR)
    z1 = jnp.zeros((NPT, D_MID // 2), jnp.float32)
    z2 = jnp.zeros((NPT, D_FIN // 2), jnp.float32)

    degp = _deg_call(dst_pad)
    g1lo, g1hi, dinv = _tc_a(x, W1, degp.T)
    acc1 = _agg_l1(g1lo, g1hi, src2d, dst2d, z1)
    g2lo, g2hi = _tc_b(acc1, g1lo, g1hi, dinv, b1, W2)
    acc2 = _agg_l2(g2lo, g2hi, src2d, dst2d, z2)
    return _tc_c(acc2, g2lo, g2hi, dinv, b2)
